# baseline (speedup 1.0000x reference)
import functools

import jax
import jax.numpy as jnp
from jax import lax
from jax.experimental import pallas as pl
from jax.experimental.pallas import tpu as pltpu
from jax.experimental.pallas import tpu_sc as plsc

DIM = 256
N = 2048
NHEADS = 8
HD = DIM // NHEADS
ALPHA = 0.2
BR = 512
NBLK = N // BR
WHX = 128

_NC = 2
_NS = 16
_NW = _NC * _NS
_BPW = N // _NW


def _gather_sc(emb, nodes):
    mesh = plsc.VectorSubcoreMesh(core_axis_name="c", subcore_axis_name="s")

    @functools.partial(
        pl.kernel,
        mesh=mesh,
        out_type=jax.ShapeDtypeStruct((N, DIM), jnp.float32),
        scratch_types=[
            pltpu.VMEM((_BPW,), jnp.int32),
            pltpu.VMEM((_BPW, DIM), jnp.float32),
            pltpu.SemaphoreType.DMA,
        ],
    )
    def k(emb_hbm, idx_hbm, out_hbm, idx_v, rows_v, sem):
        wid = lax.axis_index("s") * _NC + lax.axis_index("c")
        base = wid * _BPW
        pltpu.sync_copy(idx_hbm.at[pl.ds(base, _BPW)], idx_v)
        pltpu.async_copy(emb_hbm.at[idx_v], rows_v, sem).wait()
        pltpu.sync_copy(rows_v, out_hbm.at[pl.ds(base, _BPW)])

    return k(emb, nodes)


def _unit_setup(hv, wcat, amat, whx_s, us_s, vt_s):
    wh = jnp.dot(hv, wcat, preferred_element_type=jnp.float32)
    e = jnp.dot(wh, amat, preferred_element_type=jnp.float32)
    et = e.T
    est = et[:NHEADS, :]
    edt = et[NHEADS:, :]
    ms = jnp.max(est, axis=1, keepdims=True)
    md = jnp.max(edt, axis=1, keepdims=True)
    m = ms + md
    u1t = jnp.exp(est - ms)
    uat = jnp.exp(ALPHA * (est - ms))
    v1t = jnp.exp(edt - md)
    vat = jnp.exp(ALPHA * (edt - md) - (1.0 - ALPHA) * m)
    vt_s[...] = jnp.concatenate([v1t, vat], axis=0).astype(jnp.bfloat16)
    us_s[...] = jnp.concatenate([u1t, uat], axis=0).T.astype(jnp.bfloat16)
    whb = wh.astype(jnp.bfloat16)
    for h in range(NHEADS):
        whx_s[:, h * WHX:h * WHX + HD] = whb[:, h * HD:(h + 1) * HD]


def _adj_body(dist_ref, adj_ref):
    adj_ref[...] = jnp.where(dist_ref[...] > 0.5, 1.0, 0.0).astype(jnp.bfloat16)


def _adj_tc(dist):
    return pl.pallas_call(
        _adj_body,
        grid=(N // BR,),
        in_specs=[pl.BlockSpec((BR, N), lambda i: (i, 0))],
        out_specs=pl.BlockSpec((BR, N), lambda i: (i, 0)),
        out_shape=jax.ShapeDtypeStruct((N, N), jnp.bfloat16),
    )(dist)


def _both_units_body(h_ref, wcat_ref, amat_ref, adjb_ref, out_ref,
                     whx_s, us_s, vt_s, h1_s):
    i = pl.program_id(0)
    r = i % NBLK

    @pl.when(i == 0)
    def _setup_u1():
        one_col = jnp.ones((N, 1), jnp.bfloat16)
        for h in range(NHEADS):
            whx_s[:, h * WHX + HD:h * WHX + HD + 1] = one_col
        _unit_setup(h_ref[...], wcat_ref[0], amat_ref[0],
                    whx_s, us_s, vt_s)

    @pl.when(i == NBLK)
    def _setup_u2():
        _unit_setup(h1_s[...], wcat_ref[0], amat_ref[0],
                    whx_s, us_s, vt_s)

    adjf = adjb_ref[...]
    us = us_s[pl.ds(r * BR, BR), :]
    vt = vt_s[...]
    for h in range(NHEADS):
        u1 = us[:, h:h + 1]
        ua = us[:, NHEADS + h:NHEADS + h + 1]
        v1 = vt[h:h + 1, :]
        va = vt[NHEADS + h:NHEADS + h + 1, :]
        p = adjf * jnp.maximum(u1 * v1, ua * va)
        oz = jnp.dot(p, whx_s[:, h * WHX:(h + 1) * WHX],
                     preferred_element_type=jnp.float32)
        o = oz[:, :HD] / oz[:, HD:HD + 1]
        out_ref[:, h * HD:(h + 1) * HD] = jnp.where(o > 0, o, jnp.exp(o) - 1.0)

    @pl.when(i < NBLK)
    def _stash_u1():
        h1_s[pl.ds(r * BR, BR), :] = out_ref[...]


def _gat_tc(h, adjb, wcat2, amat2):
    return pl.pallas_call(
        _both_units_body,
        grid=(2 * NBLK,),
        in_specs=[
            pl.BlockSpec((N, DIM), lambda i: (0, 0)),
            pl.BlockSpec((1, DIM, DIM), lambda i: (i // NBLK, 0, 0)),
            pl.BlockSpec((1, DIM, 2 * NHEADS), lambda i: (i // NBLK, 0, 0)),
            pl.BlockSpec((BR, N), lambda i: (i % NBLK, 0)),
        ],
        out_specs=pl.BlockSpec((BR, DIM), lambda i: (jnp.maximum(i - NBLK, 0), 0)),
        out_shape=jax.ShapeDtypeStruct((N, DIM), jnp.float32),
        scratch_shapes=[
            pltpu.VMEM((N, NHEADS * WHX), jnp.bfloat16),
            pltpu.VMEM((N, 2 * NHEADS), jnp.bfloat16),
            pltpu.VMEM((2 * NHEADS, N), jnp.bfloat16),
            pltpu.VMEM((N, DIM), jnp.float32),
        ],
    )(h, wcat2, amat2, adjb)


def _build_weights(Ws, As):
    eye = jnp.eye(NHEADS, dtype=jnp.float32)
    wcats, amats = [], []
    for u in range(Ws.shape[0]):
        wcats.append(Ws[u].transpose(1, 0, 2).reshape(DIM, DIM))
        a_src = As[u, :, :HD, 0]
        a_dst = As[u, :, HD:, 0]
        asrc_m = (eye[:, None, :] * a_src[:, :, None]).reshape(DIM, NHEADS)
        adst_m = (eye[:, None, :] * a_dst[:, :, None]).reshape(DIM, NHEADS)
        amats.append(jnp.concatenate([asrc_m, adst_m], axis=1))
    return jnp.stack(wcats), jnp.stack(amats)


def _gat_tc_from_raw(h, dist, Ws, As):
    wcat2, amat2 = _build_weights(Ws, As)
    adjb = _adj_tc(dist)
    return _gat_tc(h, adjb, wcat2, amat2)


def kernel(nodes, dist, fied, emb, Ws, As):
    h = _gather_sc(emb, nodes)
    return _gat_tc_from_raw(h, dist, Ws, As)


# speedup vs baseline: 1.0975x; 1.0975x over previous
import functools

import jax
import jax.numpy as jnp
from jax import lax
from jax.experimental import pallas as pl
from jax.experimental.pallas import tpu as pltpu
from jax.experimental.pallas import tpu_sc as plsc

DIM = 256
N = 2048
NHEADS = 8
HD = DIM // NHEADS
ALPHA = 0.2
BR = 512
NBLK = N // BR
WHX = 128

_NC = 2
_NS = 16
_NW = _NC * _NS
_BPW = N // _NW


def _gather_sc(emb, nodes):
    mesh = plsc.VectorSubcoreMesh(core_axis_name="c", subcore_axis_name="s")

    @functools.partial(
        pl.kernel,
        mesh=mesh,
        out_type=jax.ShapeDtypeStruct((N, DIM), jnp.float32),
        scratch_types=[
            pltpu.VMEM((_BPW,), jnp.int32),
            pltpu.VMEM((_BPW, DIM), jnp.float32),
            pltpu.SemaphoreType.DMA,
        ],
    )
    def k(emb_hbm, idx_hbm, out_hbm, idx_v, rows_v, sem):
        wid = lax.axis_index("s") * _NC + lax.axis_index("c")
        base = wid * _BPW
        pltpu.sync_copy(idx_hbm.at[pl.ds(base, _BPW)], idx_v)
        pltpu.async_copy(emb_hbm.at[idx_v], rows_v, sem).wait()
        pltpu.sync_copy(rows_v, out_hbm.at[pl.ds(base, _BPW)])

    return k(emb, nodes)


def _unit_setup(hv, wcat, amat, whx_s, us_s, vt_s):
    wh = jnp.dot(hv, wcat, preferred_element_type=jnp.float32)
    e = jnp.dot(wh, amat, preferred_element_type=jnp.float32)
    et = e.T
    est = et[:NHEADS, :]
    edt = et[NHEADS:, :]
    ms = jnp.max(est, axis=1, keepdims=True)
    md = jnp.max(edt, axis=1, keepdims=True)
    m = ms + md
    u1t = jnp.exp(est - ms)
    uat = jnp.exp(ALPHA * (est - ms))
    v1t = jnp.exp(edt - md)
    vat = jnp.exp(ALPHA * (edt - md) - (1.0 - ALPHA) * m)
    vt_s[...] = jnp.concatenate([v1t, vat], axis=0).astype(jnp.bfloat16)
    us_s[...] = jnp.concatenate([u1t, uat], axis=0).T.astype(jnp.bfloat16)
    whb = wh.astype(jnp.bfloat16)
    for h in range(NHEADS):
        whx_s[:, h * WHX:h * WHX + HD] = whb[:, h * HD:(h + 1) * HD]


def _both_units_body(h_ref, wcat_ref, amat_ref, dist_ref, out_ref,
                     whx_s, us_s, vt_s, h1_s):
    i = pl.program_id(0)
    r = i % NBLK

    @pl.when(i == 0)
    def _setup_u1():
        one_col = jnp.ones((N, 1), jnp.bfloat16)
        for h in range(NHEADS):
            whx_s[:, h * WHX + HD:h * WHX + HD + 1] = one_col
        _unit_setup(h_ref[...], wcat_ref[0], amat_ref[0],
                    whx_s, us_s, vt_s)

    @pl.when(i == NBLK)
    def _setup_u2():
        _unit_setup(h1_s[...], wcat_ref[0], amat_ref[0],
                    whx_s, us_s, vt_s)

    adjf = jnp.where(dist_ref[...] > 0.5, 1.0, 0.0).astype(jnp.bfloat16)
    us = us_s[pl.ds(r * BR, BR), :]
    vt = vt_s[...]
    for h in range(NHEADS):
        u1 = us[:, h:h + 1]
        ua = us[:, NHEADS + h:NHEADS + h + 1]
        v1 = vt[h:h + 1, :]
        va = vt[NHEADS + h:NHEADS + h + 1, :]
        p = adjf * jnp.maximum(u1 * v1, ua * va)
        oz = jnp.dot(p, whx_s[:, h * WHX:(h + 1) * WHX],
                     preferred_element_type=jnp.float32)
        o = oz[:, :HD] / oz[:, HD:HD + 1]
        out_ref[:, h * HD:(h + 1) * HD] = jnp.where(o > 0, o, jnp.exp(o) - 1.0)

    @pl.when(i < NBLK)
    def _stash_u1():
        h1_s[pl.ds(r * BR, BR), :] = out_ref[...]


def _gat_tc(h, dist, wcat2, amat2):
    return pl.pallas_call(
        _both_units_body,
        grid=(2 * NBLK,),
        in_specs=[
            pl.BlockSpec((N, DIM), lambda i: (0, 0)),
            pl.BlockSpec((1, DIM, DIM), lambda i: (i // NBLK, 0, 0)),
            pl.BlockSpec((1, DIM, 2 * NHEADS), lambda i: (i // NBLK, 0, 0)),
            pl.BlockSpec((BR, N), lambda i: (i % NBLK, 0)),
        ],
        out_specs=pl.BlockSpec((BR, DIM), lambda i: (jnp.maximum(i - NBLK, 0), 0)),
        out_shape=jax.ShapeDtypeStruct((N, DIM), jnp.float32),
        scratch_shapes=[
            pltpu.VMEM((N, NHEADS * WHX), jnp.bfloat16),
            pltpu.VMEM((N, 2 * NHEADS), jnp.bfloat16),
            pltpu.VMEM((2 * NHEADS, N), jnp.bfloat16),
            pltpu.VMEM((N, DIM), jnp.float32),
        ],
    )(h, wcat2, amat2, dist)


def _build_weights(Ws, As):
    eye = jnp.eye(NHEADS, dtype=jnp.float32)
    wcats, amats = [], []
    for u in range(Ws.shape[0]):
        wcats.append(Ws[u].transpose(1, 0, 2).reshape(DIM, DIM))
        a_src = As[u, :, :HD, 0]
        a_dst = As[u, :, HD:, 0]
        asrc_m = (eye[:, None, :] * a_src[:, :, None]).reshape(DIM, NHEADS)
        adst_m = (eye[:, None, :] * a_dst[:, :, None]).reshape(DIM, NHEADS)
        amats.append(jnp.concatenate([asrc_m, adst_m], axis=1))
    return jnp.stack(wcats), jnp.stack(amats)


def _gat_tc_from_raw(h, dist, Ws, As):
    wcat2, amat2 = _build_weights(Ws, As)
    return _gat_tc(h, dist, wcat2, amat2)


def kernel(nodes, dist, fied, emb, Ws, As):
    h = _gather_sc(emb, nodes)
    return _gat_tc_from_raw(h, dist, Ws, As)
